# baseline (speedup 1.0000x reference)
import jax
import jax.numpy as jnp
from jax.experimental import pallas as pl
from jax.experimental.pallas import tpu as pltpu

_EPS = 1e-05
_S = 8
_BN = 512


def _bn_kernel(x_ref, m_ref, g_ref, b_ref, mu_ref, var_ref, o_ref):
    inv = jax.lax.rsqrt(var_ref[...] + _EPS)
    scale = g_ref[...] * inv
    bias = b_ref[...] - mu_ref[...] * scale
    x = x_ref[...]
    m = m_ref[...]
    normed = x * scale + bias
    o_ref[...] = x + m * (normed - x)


def kernel(x_flat_nc, mask_flat, gamma, beta, moving_mean, moving_var):
    n, c = x_flat_nc.shape
    rows = n // _S
    x3 = x_flat_nc.reshape(_S, rows, c)
    m3 = mask_flat.astype(jnp.float32).reshape(_S, rows, 1)
    grid = (rows // _BN,)
    out = pl.pallas_call(
        _bn_kernel,
        grid=grid,
        in_specs=[
            pl.BlockSpec((_S, _BN, c), lambda i: (0, i, 0)),
            pl.BlockSpec((_S, _BN, 1), lambda i: (0, i, 0)),
            pl.BlockSpec((1, 1, c), lambda i: (0, 0, 0)),
            pl.BlockSpec((1, 1, c), lambda i: (0, 0, 0)),
            pl.BlockSpec((1, 1, c), lambda i: (0, 0, 0)),
            pl.BlockSpec((1, 1, c), lambda i: (0, 0, 0)),
        ],
        out_specs=pl.BlockSpec((_S, _BN, c), lambda i: (0, i, 0)),
        out_shape=jax.ShapeDtypeStruct((_S, rows, c), x_flat_nc.dtype),
        compiler_params=pltpu.CompilerParams(
            dimension_semantics=("parallel",),
        ),
    )(x3, m3, gamma[None, None, :], beta[None, None, :],
      moving_mean[None, None, :], moving_var[None, None, :])
    return out.reshape(n, c)
